# Initial kernel scaffold; baseline (speedup 1.0000x reference)
import jax
import jax.numpy as jnp
from jax.experimental import pallas as pl


def kernel(image_tensor, depth, meter_per_pixel, sat_width):
    raise NotImplementedError("write your pallas kernel here")


# speedup vs baseline: 13.3542x; 13.3542x over previous
import numpy as np

import jax
import jax.numpy as jnp
from jax import lax
from jax.experimental import pallas as pl
from jax.experimental.pallas import tpu as pltpu
from jax.experimental.pallas import tpu_sc as plsc

B, C, H, W = 8, 64, 160, 320
HW = H * W
GRID = 128
NCELL = GRID * GRID
QP = HW // 4
HQ = QP // 4
L = 16
QT = NCELL // 4
NEG_INF = float("-inf")


def _sc_mesh():
    return plsc.VectorSubcoreMesh(core_axis_name="c", subcore_axis_name="s",
                                  num_cores=2, num_subcores=16)


def _build_geometry():
    theta = jnp.linspace(0.0, 2.0 * jnp.pi, W)
    phi = jnp.linspace(0.0, jnp.pi, H)
    phi2, theta2 = jnp.meshgrid(phi, theta, indexing="ij")
    x = jnp.sin(phi2) * jnp.cos(theta2)
    y = -jnp.cos(phi2)
    z = -jnp.sin(phi2) * jnp.sin(theta2)
    xyz = jnp.stack((x, y, z), axis=-1).astype(jnp.float32)
    rot = jnp.array([[0.0, 0.0, 1.0], [0.0, 1.0, 0.0], [-1.0, 0.0, 0.0]],
                    dtype=jnp.float32)
    xyz = xyz.reshape(-1, 3) @ rot.T
    return (np.asarray(xyz[:, 0]).reshape(1, HW),
            np.asarray(xyz[:, 1]).reshape(1, HW),
            np.asarray(xyz[:, 2]).reshape(1, HW))


_XS, _YS, _ZS = _build_geometry()


def _tc_prepass(dep_ref, xs_ref, ys_ref, zs_ref, mpp_ref, cell_ref, y2_ref):
    dep = dep_ref[...]
    xs = xs_ref[...]
    ys = ys_ref[...]
    zs = zs_ref[...]
    mpp = mpp_ref[...][:, 0:1]
    xq = (xs * dep) / mpp
    zq = (zs * dep) / mpp
    xi = xq.astype(jnp.int32)
    zi = zq.astype(jnp.int32)
    yg = ys * dep
    kept = (xi >= -64) & (xi <= 63) & (zi >= -64) & (zi <= 63)
    maxh = jnp.max(jnp.where(kept, yg, NEG_INF))
    y2 = maxh - yg
    cell = (xi + 64) * GRID + (zi + 64)
    cell_ref[...] = jnp.where(kept, cell, -1)
    y2_ref[...] = jnp.where(kept, y2, NEG_INF)


def _tc_flatten(img_ref, out_ref):
    out_ref[...] = img_ref[0].reshape(8, HW)


def _sc_all(cell_hbm, y2_hbm, img_hbm, out_hbm,
            cellv, y2v, tby, tbp, iny, inp_, plane, outb,
            shy, shp, sem):
    cid = lax.axis_index("c")
    sid = lax.axis_index("s")
    wid = cid * 16 + sid
    b = wid // 4
    q = wid % 4

    ninf = jnp.full((L,), NEG_INF, jnp.float32)
    mone = jnp.full((L,), -1, jnp.int32)

    def init_body(i, carry):
        tby[pl.ds(i * L, L)] = ninf
        tbp[pl.ds(i * L, L)] = mone
        return carry

    lax.fori_loop(0, NCELL // L, init_body, 0)

    iot = lax.iota(jnp.int32, L)

    def half_body(hh, carry):
        base = b * HW + q * QP + hh * HQ
        pltpu.async_copy(cell_hbm.at[pl.ds(base, HQ)], cellv, sem).wait()
        pltpu.async_copy(y2_hbm.at[pl.ds(base, HQ)], y2v, sem).wait()
        pbase = q * QP + hh * HQ

        def pt_body(i, carry2):
            cells = cellv[pl.ds(i * L, L)]
            y2l = y2v[pl.ds(i * L, L)]
            pidx = pbase + i * L + iot
            mask0 = cells >= 0
            cells_safe = jnp.where(mask0, cells, 0)

            def wcond(rem):
                return jnp.max(rem.astype(jnp.int32)) > 0

            def wbody(rem):
                oy = plsc.load_gather(tby, [cells_safe])
                op = plsc.load_gather(tbp, [cells_safe])
                better = rem & ((y2l > oy) | ((y2l == oy) & (pidx > op)))
                plsc.store_scatter(tby, [cells_safe], y2l, mask=better)
                plsc.store_scatter(tbp, [cells_safe], pidx, mask=better)
                chk = plsc.load_gather(tbp, [cells_safe])
                return better & (chk != pidx)

            lax.while_loop(wcond, wbody, mask0)
            return carry2

        lax.fori_loop(0, HQ // L, pt_body, 0)
        return carry

    lax.fori_loop(0, 4, half_body, 0)

    pltpu.sync_copy(tby, shy.at[wid])
    pltpu.sync_copy(tbp, shp.at[wid])
    plsc.subcore_barrier()

    g0 = (sid // 4) * 4
    for k in (1, 2, 3):
        partner = cid * 16 + g0 + ((sid - g0 + k) % 4)
        for hq in (0, 1, 2, 3):
            pltpu.async_copy(shy.at[partner, pl.ds(hq * QT, QT)], iny,
                             sem).wait()
            pltpu.async_copy(shp.at[partner, pl.ds(hq * QT, QT)], inp_,
                             sem).wait()

            def mbody(j, carry, _hq=hq):
                off = _hq * QT + j * L
                ay = tby[pl.ds(off, L)]
                ap = tbp[pl.ds(off, L)]
                ny = iny[pl.ds(j * L, L)]
                np_ = inp_[pl.ds(j * L, L)]
                take = (ny > ay) | ((ny == ay) & (np_ > ap))
                tby[pl.ds(off, L)] = jnp.where(take, ny, ay)
                tbp[pl.ds(off, L)] = jnp.where(take, np_, ap)
                return carry

            lax.fori_loop(0, QT // L, mbody, 0)

    def idx_body(j, carry):
        pv = tbp[pl.ds(j * L, L)]
        tbp[pl.ds(j * L, L)] = jnp.where(pv < 0, HW, pv)
        return carry

    lax.fori_loop(0, NCELL // L, idx_body, 0)

    plane[pl.ds(HW, L)] = jnp.zeros((L,), jnp.float32)

    def ch_body(ch, carry):
        cidx = q * 16 + ch
        pltpu.async_copy(img_hbm.at[b * C + cidx, pl.ds(0, HW)],
                         plane.at[pl.ds(0, HW)], sem).wait()

        def gbody(j, carry2):
            idxv = tbp[pl.ds(j * L, L)]
            outb[pl.ds(j * L, L)] = plsc.load_gather(plane, [idxv])
            return carry2

        lax.fori_loop(0, NCELL // L, gbody, 0)
        ooff = (b * C + cidx) * NCELL
        pltpu.async_copy(outb, out_hbm.at[pl.ds(ooff, NCELL)], sem).wait()
        return carry

    lax.fori_loop(0, 16, ch_body, 0)


def kernel(image_tensor, depth, meter_per_pixel, sat_width):
    del sat_width

    xs = jnp.asarray(_XS)
    ys = jnp.asarray(_YS)
    zs = jnp.asarray(_ZS)

    dep = depth.reshape(B, HW)
    mpp = jnp.broadcast_to(meter_per_pixel.reshape(B, 1), (B, 128))

    cell, y2 = pl.pallas_call(
        _tc_prepass,
        out_shape=(
            jax.ShapeDtypeStruct((B, HW), jnp.int32),
            jax.ShapeDtypeStruct((B, HW), jnp.float32),
        ),
    )(dep, xs, ys, zs, mpp)

    img_flat = pl.pallas_call(
        _tc_flatten,
        grid=(B * C // 8,),
        in_specs=[pl.BlockSpec((1, 8, H, W), lambda j: (j // 8, j % 8, 0, 0))],
        out_specs=pl.BlockSpec((8, HW), lambda j: (j, 0)),
        out_shape=jax.ShapeDtypeStruct((B * C, HW), jnp.float32),
    )(image_tensor)

    sc_all = pl.kernel(
        _sc_all,
        out_type=jax.ShapeDtypeStruct((B * C * NCELL,), jnp.float32),
        mesh=_sc_mesh(),
        scratch_types=[
            pltpu.VMEM((HQ,), jnp.int32),
            pltpu.VMEM((HQ,), jnp.float32),
            pltpu.VMEM((NCELL,), jnp.float32),
            pltpu.VMEM((NCELL,), jnp.int32),
            pltpu.VMEM((QT,), jnp.float32),
            pltpu.VMEM((QT,), jnp.int32),
            pltpu.VMEM((HW + L,), jnp.float32),
            pltpu.VMEM((NCELL,), jnp.float32),
            pltpu.HBM((32, NCELL), jnp.float32),
            pltpu.HBM((32, NCELL), jnp.int32),
            pltpu.SemaphoreType.DMA,
        ],
        compiler_params=pltpu.CompilerParams(needs_layout_passes=False),
    )
    out = sc_all(cell.reshape(B * HW), y2.reshape(B * HW), img_flat)
    return out.reshape(B, C, GRID, GRID)
